# Initial kernel scaffold; baseline (speedup 1.0000x reference)
import jax
import jax.numpy as jnp
from jax.experimental import pallas as pl


def kernel(x, pos, vel, edge_index, edge_attr, params):
    raise NotImplementedError("write your pallas kernel here")


# speedup vs baseline: 1.2612x; 1.2612x over previous
import functools

import jax
import jax.numpy as jnp
from jax import lax
from jax.experimental import pallas as pl
from jax.experimental.pallas import tpu as pltpu

T, N, D = 4, 10000, 128
E = 320000
D_EDGE = 16
POS = 3
HID = 64
NTOT = T * N

F32 = jnp.float32


def _silu(v):
    return v / (1.0 + jnp.exp(-v))


def _dot(a, b):
    return jnp.dot(a, b, preferred_element_type=F32)


def _spectral_x_body(x_ref, w0r_ref, w1r_ref, w1i_ref, wd_ref, wq_ref,
                     x2_ref, p_ref, q_ref):
    x0 = x_ref[0]
    x1 = x_ref[1]
    x2 = x_ref[2]
    x3 = x_ref[3]
    f0 = x0 + x1 + x2 + x3
    a = x0 - x2
    b = x3 - x1
    r0 = _dot(f0, w0r_ref[...])
    r1 = _dot(a, w1r_ref[...]) - _dot(b, w1i_ref[...])
    i1 = _dot(a, w1i_ref[...]) + _dot(b, w1r_ref[...])
    y0 = 0.25 * (r0 + 2.0 * r1)
    y1 = 0.25 * (r0 - 2.0 * i1)
    y2 = 0.25 * (r0 - 2.0 * r1)
    y3 = 0.25 * (r0 + 2.0 * i1)
    o0 = x0 + y0
    o1 = x1 + y1
    o2 = x2 + y2
    o3 = x3 + y3
    x2_ref[0] = o0
    x2_ref[1] = o1
    x2_ref[2] = o2
    x2_ref[3] = o3
    wd = wd_ref[...]
    wq = wq_ref[...]
    p_ref[0] = _dot(o0, wd)
    p_ref[1] = _dot(o1, wd)
    p_ref[2] = _dot(o2, wd)
    p_ref[3] = _dot(o3, wd)
    q_ref[0] = _dot(o0, wq)
    q_ref[1] = _dot(o1, wq)
    q_ref[2] = _dot(o2, wq)
    q_ref[3] = _dot(o3, wq)


def _spectral_x(x, w0r, w1r, w1i, wd, wq):
    nb = 1000
    grid = N // nb
    full = lambda shape: pl.BlockSpec(shape, lambda i: (0,) * len(shape))
    return pl.pallas_call(
        _spectral_x_body,
        grid=(grid,),
        in_specs=[
            pl.BlockSpec((T, nb, D), lambda i: (0, i, 0)),
            full((D, D)), full((D, D)), full((D, D)),
            full((D, HID)), full((D, HID)),
        ],
        out_specs=[
            pl.BlockSpec((T, nb, D), lambda i: (0, i, 0)),
            pl.BlockSpec((T, nb, HID), lambda i: (0, i, 0)),
            pl.BlockSpec((T, nb, HID), lambda i: (0, i, 0)),
        ],
        out_shape=[
            jax.ShapeDtypeStruct((T, N, D), F32),
            jax.ShapeDtypeStruct((T, N, HID), F32),
            jax.ShapeDtypeStruct((T, N, HID), F32),
        ],
    )(x, w0r, w1r, w1i, wd, wq)


def _center_body(pos_ref, out_ref):
    i = pl.program_id(0)

    @pl.when(i == 0)
    def _init():
        out_ref[...] = jnp.zeros_like(out_ref)

    part = jnp.sum(pos_ref[...], axis=1)
    out_ref[...] += part * (1.0 / N)


def _center(pos):
    nb = 1000
    return pl.pallas_call(
        _center_body,
        grid=(N // nb,),
        in_specs=[pl.BlockSpec((T, nb, POS), lambda i: (0, i, 0))],
        out_specs=pl.BlockSpec((T, POS), lambda i: (0, 0)),
        out_shape=jax.ShapeDtypeStruct((T, POS), F32),
    )(pos)


def _spectral_v_body(pos_ref, vel_ref, c_ref, wr_ref, wi_ref, pf_ref, vf_ref):
    center = []
    pc = []
    vv = []
    for t in range(T):
        ct = c_ref[t]
        center.append(ct)
        pc.append(pos_ref[t] - ct)
        vv.append(vel_ref[t])
    f0p = pc[0] + pc[1] + pc[2] + pc[3]
    f0v = vv[0] + vv[1] + vv[2] + vv[3]
    ap = pc[0] - pc[2]
    av = vv[0] - vv[2]
    bp = pc[3] - pc[1]
    bv = vv[3] - vv[1]

    def mix(arr_p, arr_v, w_ref, o, m):
        return arr_p * w_ref[4 * 0 + 2 * o + m] + arr_v * w_ref[4 * 1 + 2 * o + m]

    for o in range(2):
        r0 = mix(f0p, f0v, wr_ref, o, 0)
        r1 = mix(ap, av, wr_ref, o, 1) - mix(bp, bv, wi_ref, o, 1)
        i1 = mix(ap, av, wi_ref, o, 1) + mix(bp, bv, wr_ref, o, 1)
        y = (0.25 * (r0 + 2.0 * r1), 0.25 * (r0 - 2.0 * i1),
             0.25 * (r0 - 2.0 * r1), 0.25 * (r0 + 2.0 * i1))
        for t in range(T):
            if o == 0:
                pf_ref[t, :, 0:POS] = pc[t] + y[t] + center[t]
            else:
                vf_ref[t, :, 0:POS] = vv[t] + y[t]
    nb = pf_ref.shape[1]
    pf_ref[:, :, POS:] = jnp.zeros((T, nb, 16 - POS), F32)
    vf_ref[:, :, POS:] = jnp.zeros((T, nb, 16 - POS), F32)


def _spectral_v(pos, vel, center, wr8, wi8):
    nb = 1000
    return pl.pallas_call(
        _spectral_v_body,
        grid=(N // nb,),
        in_specs=[
            pl.BlockSpec((T, nb, POS), lambda i: (0, i, 0)),
            pl.BlockSpec((T, nb, POS), lambda i: (0, i, 0)),
            pl.BlockSpec((T, POS), lambda i: (0, 0)),
            pl.BlockSpec(memory_space=pltpu.SMEM),
            pl.BlockSpec(memory_space=pltpu.SMEM),
        ],
        out_specs=[
            pl.BlockSpec((T, nb, 16), lambda i: (0, i, 0)),
            pl.BlockSpec((T, nb, 16), lambda i: (0, i, 0)),
        ],
        out_shape=[
            jax.ShapeDtypeStruct((T, N, 16), F32),
            jax.ShapeDtypeStruct((T, N, 16), F32),
        ],
    )(pos, vel, center, wr8, wi8)


def _edge_body(h_ref, diff_ref, ea_ref, wdist_ref, wea_ref, b1_ref,
               w2_ref, b2_ref, w3_ref, b3_ref,
               v1_ref, c1_ref, v2_ref, c2_ref, v3_ref, c3_ref,
               out_ref):
    diff = diff_ref[...]
    dist = jnp.sqrt(jnp.sum(diff * diff, axis=1, keepdims=True) + 1e-12)
    u = h_ref[...] + _dot(ea_ref[...], wea_ref[...]) \
        + dist * wdist_ref[...] + b1_ref[...]
    u = _silu(u)
    u = _silu(_dot(u, w2_ref[...]) + b2_ref[...])
    m = _dot(u, w3_ref[...]) + b3_ref[...]
    p = _silu(_dot(m, v1_ref[...]) + c1_ref[...])
    p = _silu(_dot(p, v2_ref[...]) + c2_ref[...])
    s = _dot(p, v3_ref[...]) + c3_ref[...]
    out_ref[:, 0:HID] = m
    out_ref[:, HID:HID + 16] = diff * s


def _edge_mlp(h, diff16, ea, wdist, wea, b1, w2, b2, w3, b3,
              v1, c1, v2, c2, v3, c3):
    eb = 3200
    grid = E // eb
    full = lambda shape: pl.BlockSpec(shape, lambda i: (0,) * len(shape))
    return pl.pallas_call(
        _edge_body,
        grid=(grid,),
        in_specs=[
            pl.BlockSpec((eb, HID), lambda i: (i, 0)),
            pl.BlockSpec((eb, 16), lambda i: (i, 0)),
            pl.BlockSpec((eb, D_EDGE), lambda i: (i, 0)),
            full((1, HID)), full((D_EDGE, HID)), full((1, HID)),
            full((HID, HID)), full((1, HID)),
            full((HID, HID)), full((1, HID)),
            full((HID, HID)), full((1, HID)),
            full((HID, HID)), full((1, HID)),
            full((HID, 1)), full((1, 1)),
        ],
        out_specs=pl.BlockSpec((eb, HID + 16), lambda i: (i, 0)),
        out_shape=jax.ShapeDtypeStruct((E, HID + 16), F32),
    )(h, diff16, ea, wdist, wea, b1, w2, b2, w3, b3, v1, c1, v2, c2, v3, c3)


def _node_body(xf_ref, ag_ref, pf_ref, vf_ref,
               u1a_ref, u1b_ref, fb1_ref, u2_ref, fb2_ref, u3_ref, fb3_ref,
               z1_ref, zb1_ref, z2_ref, zb2_ref, z3_ref, zb3_ref,
               xn_ref, pn_ref, vn_ref):
    xf = xf_ref[...]
    ag = ag_ref[...]
    am = ag[:, 0:HID]
    ap = ag[:, HID:HID + 16]
    h = _silu(_dot(xf, u1a_ref[...]) + _dot(am, u1b_ref[...]) + fb1_ref[...])
    h = _silu(_dot(h, u2_ref[...]) + fb2_ref[...])
    xn_ref[...] = _dot(h, u3_ref[...]) + fb3_ref[...]
    z = _silu(_dot(xf, z1_ref[...]) + zb1_ref[...])
    z = _silu(_dot(z, z2_ref[...]) + zb2_ref[...])
    s = _dot(z, z3_ref[...]) + zb3_ref[...]
    vn = s * vf_ref[...] + ap
    vn_ref[...] = vn
    pn_ref[...] = pf_ref[...] + vn


def _node_update(xf, ag, pf16, vf16, u1a, u1b, fb1, u2, fb2, u3, fb3,
                 z1, zb1, z2, zb2, z3, zb3):
    nb = 2000
    grid = NTOT // nb
    full = lambda shape: pl.BlockSpec(shape, lambda i: (0,) * len(shape))
    return pl.pallas_call(
        _node_body,
        grid=(grid,),
        in_specs=[
            pl.BlockSpec((nb, D), lambda i: (i, 0)),
            pl.BlockSpec((nb, HID + 16), lambda i: (i, 0)),
            pl.BlockSpec((nb, 16), lambda i: (i, 0)),
            pl.BlockSpec((nb, 16), lambda i: (i, 0)),
            full((D, HID)), full((HID, HID)), full((1, HID)),
            full((HID, HID)), full((1, HID)),
            full((HID, D)), full((1, D)),
            full((D, HID)), full((1, HID)),
            full((HID, HID)), full((1, HID)),
            full((HID, 1)), full((1, 1)),
        ],
        out_specs=[
            pl.BlockSpec((nb, D), lambda i: (i, 0)),
            pl.BlockSpec((nb, 16), lambda i: (i, 0)),
            pl.BlockSpec((nb, 16), lambda i: (i, 0)),
        ],
        out_shape=[
            jax.ShapeDtypeStruct((NTOT, D), F32),
            jax.ShapeDtypeStruct((NTOT, 16), F32),
            jax.ShapeDtypeStruct((NTOT, 16), F32),
        ],
    )(xf, ag, pf16, vf16, u1a, u1b, fb1, u2, fb2, u3, fb3,
      z1, zb1, z2, zb2, z3, zb3)


def _gather_stage(p2, q2, pf16f, src, dst):
    h = jnp.take(p2, dst, axis=0) + jnp.take(q2, src, axis=0)
    diff16 = jnp.take(pf16f, dst, axis=0) - jnp.take(pf16f, src, axis=0)
    return h, diff16


def _scatter_stage(mp, dst):
    return jax.ops.segment_sum(mp, dst, num_segments=NTOT)


def kernel(x, pos, vel, edge_index, edge_attr, params):
    w0r = params['weight_scalar_r'][:, :, 0]
    w1r = params['weight_scalar_r'][:, :, 1]
    w1i = params['weight_scalar_i'][:, :, 1]
    wvr8 = params['weight_vector_r'].reshape(8)
    wvi8 = params['weight_vector_i'].reshape(8)

    (mw1, mb1), (mw2, mb2), (mw3, mb3) = params['message_net']
    wd = mw1[0:D]
    wq = mw1[D:2 * D]
    wdist = mw1[2 * D:2 * D + 1]
    wea = mw1[2 * D + 1:]

    (pv1, pc1), (pv2, pc2), (pv3, pc3) = params['update_pos_net']
    (fu1, fb1), (fu2, fb2), (fu3, fb3) = params['update_feat_net']
    u1a = fu1[0:D]
    u1b = fu1[D:]
    (zv1, zb1), (zv2, zb2), (zv3, zb3) = params['update_vel_net']

    row = lambda b: b.reshape(1, -1)

    x2, p, q = _spectral_x(x, w0r, w1r, w1i, wd, wq)
    center = _center(pos)
    pf16, vf16 = _spectral_v(pos, vel, center, wvr8, wvi8)

    xf = x2.reshape(NTOT, D)
    p2 = p.reshape(NTOT, HID)
    q2 = q.reshape(NTOT, HID)
    pf16f = pf16.reshape(NTOT, 16)
    vf16f = vf16.reshape(NTOT, 16)
    src = edge_index[0]
    dst = edge_index[1]
    ea = edge_attr.reshape(E, D_EDGE)

    h, diff16 = _gather_stage(p2, q2, pf16f, src, dst)

    mp = _edge_mlp(h, diff16, ea, row(wdist.reshape(-1)), wea, row(mb1),
                   mw2, row(mb2), mw3, row(mb3),
                   pv1, row(pc1), pv2, row(pc2), pv3, row(pc3))

    ag = _scatter_stage(mp, dst)

    xn, pn16, vn16 = _node_update(
        xf, ag, pf16f, vf16f,
        u1a, u1b, row(fb1), fu2, row(fb2), fu3, row(fb3),
        zv1, row(zb1), zv2, row(zb2), zv3, row(zb3))

    x_new = xn.reshape(T, N, D)
    pos_new = pn16[:, 0:POS].reshape(T, N, POS)
    vel_new = vn16[:, 0:POS].reshape(T, N, POS)
    return (x_new, pos_new, vel_new)


# speedup vs baseline: 2.6103x; 2.0698x over previous
import functools

import jax
import jax.numpy as jnp
from jax import lax
from jax.experimental import pallas as pl
from jax.experimental.pallas import tpu as pltpu
from jax.experimental.pallas import tpu_sc as plsc

T, N, D = 4, 10000, 128
E = 320000
D_EDGE = 16
POS = 3
HID = 64
NTOT = T * N

F32 = jnp.float32


def _silu(v):
    return v / (1.0 + jnp.exp(-v))


def _dot(a, b):
    return jnp.dot(a, b, preferred_element_type=F32)


def _spectral_x_body(x_ref, w0r_ref, w1r_ref, w1i_ref, wd_ref, wq_ref,
                     x2_ref, p_ref, q_ref):
    x0 = x_ref[0]
    x1 = x_ref[1]
    x2 = x_ref[2]
    x3 = x_ref[3]
    f0 = x0 + x1 + x2 + x3
    a = x0 - x2
    b = x3 - x1
    r0 = _dot(f0, w0r_ref[...])
    r1 = _dot(a, w1r_ref[...]) - _dot(b, w1i_ref[...])
    i1 = _dot(a, w1i_ref[...]) + _dot(b, w1r_ref[...])
    y0 = 0.25 * (r0 + 2.0 * r1)
    y1 = 0.25 * (r0 - 2.0 * i1)
    y2 = 0.25 * (r0 - 2.0 * r1)
    y3 = 0.25 * (r0 + 2.0 * i1)
    o0 = x0 + y0
    o1 = x1 + y1
    o2 = x2 + y2
    o3 = x3 + y3
    x2_ref[0] = o0
    x2_ref[1] = o1
    x2_ref[2] = o2
    x2_ref[3] = o3
    wd = wd_ref[...]
    wq = wq_ref[...]
    p_ref[0] = _dot(o0, wd)
    p_ref[1] = _dot(o1, wd)
    p_ref[2] = _dot(o2, wd)
    p_ref[3] = _dot(o3, wd)
    q_ref[0] = _dot(o0, wq)
    q_ref[1] = _dot(o1, wq)
    q_ref[2] = _dot(o2, wq)
    q_ref[3] = _dot(o3, wq)


def _spectral_x(x, w0r, w1r, w1i, wd, wq):
    nb = 1000
    grid = N // nb
    full = lambda shape: pl.BlockSpec(shape, lambda i: (0,) * len(shape))
    return pl.pallas_call(
        _spectral_x_body,
        grid=(grid,),
        in_specs=[
            pl.BlockSpec((T, nb, D), lambda i: (0, i, 0)),
            full((D, D)), full((D, D)), full((D, D)),
            full((D, HID)), full((D, HID)),
        ],
        out_specs=[
            pl.BlockSpec((T, nb, D), lambda i: (0, i, 0)),
            pl.BlockSpec((T, nb, HID), lambda i: (0, i, 0)),
            pl.BlockSpec((T, nb, HID), lambda i: (0, i, 0)),
        ],
        out_shape=[
            jax.ShapeDtypeStruct((T, N, D), F32),
            jax.ShapeDtypeStruct((T, N, HID), F32),
            jax.ShapeDtypeStruct((T, N, HID), F32),
        ],
    )(x, w0r, w1r, w1i, wd, wq)


def _center_body(pos_ref, out_ref):
    i = pl.program_id(0)

    @pl.when(i == 0)
    def _init():
        out_ref[...] = jnp.zeros_like(out_ref)

    part = jnp.sum(pos_ref[...], axis=1)
    out_ref[...] += part * (1.0 / N)


def _center(pos):
    nb = 1000
    return pl.pallas_call(
        _center_body,
        grid=(N // nb,),
        in_specs=[pl.BlockSpec((T, nb, POS), lambda i: (0, i, 0))],
        out_specs=pl.BlockSpec((T, POS), lambda i: (0, 0)),
        out_shape=jax.ShapeDtypeStruct((T, POS), F32),
    )(pos)


def _spectral_v_body(pos_ref, vel_ref, c_ref, wr_ref, wi_ref, pf_ref, vf_ref):
    center = []
    pc = []
    vv = []
    for t in range(T):
        ct = c_ref[t]
        center.append(ct)
        pc.append(pos_ref[t] - ct)
        vv.append(vel_ref[t])
    f0p = pc[0] + pc[1] + pc[2] + pc[3]
    f0v = vv[0] + vv[1] + vv[2] + vv[3]
    ap = pc[0] - pc[2]
    av = vv[0] - vv[2]
    bp = pc[3] - pc[1]
    bv = vv[3] - vv[1]

    def mix(arr_p, arr_v, w_ref, o, m):
        return arr_p * w_ref[4 * 0 + 2 * o + m] + arr_v * w_ref[4 * 1 + 2 * o + m]

    for o in range(2):
        r0 = mix(f0p, f0v, wr_ref, o, 0)
        r1 = mix(ap, av, wr_ref, o, 1) - mix(bp, bv, wi_ref, o, 1)
        i1 = mix(ap, av, wi_ref, o, 1) + mix(bp, bv, wr_ref, o, 1)
        y = (0.25 * (r0 + 2.0 * r1), 0.25 * (r0 - 2.0 * i1),
             0.25 * (r0 - 2.0 * r1), 0.25 * (r0 + 2.0 * i1))
        for t in range(T):
            if o == 0:
                pf_ref[t, :, 0:POS] = pc[t] + y[t] + center[t]
            else:
                vf_ref[t, :, 0:POS] = vv[t] + y[t]
    nb = pf_ref.shape[1]
    pf_ref[:, :, POS:] = jnp.zeros((T, nb, 16 - POS), F32)
    vf_ref[:, :, POS:] = jnp.zeros((T, nb, 16 - POS), F32)


def _spectral_v(pos, vel, center, wr8, wi8):
    nb = 1000
    return pl.pallas_call(
        _spectral_v_body,
        grid=(N // nb,),
        in_specs=[
            pl.BlockSpec((T, nb, POS), lambda i: (0, i, 0)),
            pl.BlockSpec((T, nb, POS), lambda i: (0, i, 0)),
            pl.BlockSpec((T, POS), lambda i: (0, 0)),
            pl.BlockSpec(memory_space=pltpu.SMEM),
            pl.BlockSpec(memory_space=pltpu.SMEM),
        ],
        out_specs=[
            pl.BlockSpec((T, nb, 16), lambda i: (0, i, 0)),
            pl.BlockSpec((T, nb, 16), lambda i: (0, i, 0)),
        ],
        out_shape=[
            jax.ShapeDtypeStruct((T, N, 16), F32),
            jax.ShapeDtypeStruct((T, N, 16), F32),
        ],
    )(pos, vel, center, wr8, wi8)


def _edge_body(h_ref, diff_ref, ea_ref, wdist_ref, wea_ref, b1_ref,
               w2_ref, b2_ref, w3_ref, b3_ref,
               v1_ref, c1_ref, v2_ref, c2_ref, v3_ref, c3_ref,
               out_ref):
    diff = diff_ref[...]
    dist = jnp.sqrt(jnp.sum(diff * diff, axis=1, keepdims=True) + 1e-12)
    u = h_ref[...] + _dot(ea_ref[...], wea_ref[...]) \
        + dist * wdist_ref[...] + b1_ref[...]
    u = _silu(u)
    u = _silu(_dot(u, w2_ref[...]) + b2_ref[...])
    m = _dot(u, w3_ref[...]) + b3_ref[...]
    p = _silu(_dot(m, v1_ref[...]) + c1_ref[...])
    p = _silu(_dot(p, v2_ref[...]) + c2_ref[...])
    s = _dot(p, v3_ref[...]) + c3_ref[...]
    out_ref[:, 0:HID] = m
    out_ref[:, HID:HID + 16] = diff * s


def _edge_mlp(h, diff16, ea, wdist, wea, b1, w2, b2, w3, b3,
              v1, c1, v2, c2, v3, c3):
    eb = 3200
    grid = E // eb
    full = lambda shape: pl.BlockSpec(shape, lambda i: (0,) * len(shape))
    return pl.pallas_call(
        _edge_body,
        grid=(grid,),
        in_specs=[
            pl.BlockSpec((eb, HID), lambda i: (i, 0)),
            pl.BlockSpec((eb, 16), lambda i: (i, 0)),
            pl.BlockSpec((eb, D_EDGE), lambda i: (i, 0)),
            full((1, HID)), full((D_EDGE, HID)), full((1, HID)),
            full((HID, HID)), full((1, HID)),
            full((HID, HID)), full((1, HID)),
            full((HID, HID)), full((1, HID)),
            full((HID, HID)), full((1, HID)),
            full((HID, 1)), full((1, 1)),
        ],
        out_specs=pl.BlockSpec((eb, HID + 16), lambda i: (i, 0)),
        out_shape=jax.ShapeDtypeStruct((E, HID + 16), F32),
    )(h, diff16, ea, wdist, wea, b1, w2, b2, w3, b3, v1, c1, v2, c2, v3, c3)


def _node_body(xf_ref, ag_ref, pf_ref, vf_ref,
               u1a_ref, u1b_ref, fb1_ref, u2_ref, fb2_ref, u3_ref, fb3_ref,
               z1_ref, zb1_ref, z2_ref, zb2_ref, z3_ref, zb3_ref,
               xn_ref, pn_ref, vn_ref):
    xf = xf_ref[...]
    ag = ag_ref[...]
    am = ag[:, 0:HID]
    ap = ag[:, HID:HID + 16]
    h = _silu(_dot(xf, u1a_ref[...]) + _dot(am, u1b_ref[...]) + fb1_ref[...])
    h = _silu(_dot(h, u2_ref[...]) + fb2_ref[...])
    xn_ref[...] = _dot(h, u3_ref[...]) + fb3_ref[...]
    z = _silu(_dot(xf, z1_ref[...]) + zb1_ref[...])
    z = _silu(_dot(z, z2_ref[...]) + zb2_ref[...])
    s = _dot(z, z3_ref[...]) + zb3_ref[...]
    vn = s * vf_ref[...] + ap
    vn_ref[...] = vn
    pn_ref[...] = pf_ref[...] + vn


def _node_update(xf, ag, pf16, vf16, u1a, u1b, fb1, u2, fb2, u3, fb3,
                 z1, zb1, z2, zb2, z3, zb3):
    nb = 2000
    grid = NTOT // nb
    full = lambda shape: pl.BlockSpec(shape, lambda i: (0,) * len(shape))
    return pl.pallas_call(
        _node_body,
        grid=(grid,),
        in_specs=[
            pl.BlockSpec((nb, D), lambda i: (i, 0)),
            pl.BlockSpec((nb, HID + 16), lambda i: (i, 0)),
            pl.BlockSpec((nb, 16), lambda i: (i, 0)),
            pl.BlockSpec((nb, 16), lambda i: (i, 0)),
            full((D, HID)), full((HID, HID)), full((1, HID)),
            full((HID, HID)), full((1, HID)),
            full((HID, D)), full((1, D)),
            full((D, HID)), full((1, HID)),
            full((HID, HID)), full((1, HID)),
            full((HID, 1)), full((1, 1)),
        ],
        out_specs=[
            pl.BlockSpec((nb, D), lambda i: (i, 0)),
            pl.BlockSpec((nb, 16), lambda i: (i, 0)),
            pl.BlockSpec((nb, 16), lambda i: (i, 0)),
        ],
        out_shape=[
            jax.ShapeDtypeStruct((NTOT, D), F32),
            jax.ShapeDtypeStruct((NTOT, 16), F32),
            jax.ShapeDtypeStruct((NTOT, 16), F32),
        ],
    )(xf, ag, pf16, vf16, u1a, u1b, fb1, u2, fb2, u3, fb3,
      z1, zb1, z2, zb2, z3, zb3)


_NC = 2
_NS = 16
_NW = _NC * _NS
_GC = 80
_EPW = E // _NW
_GNCH = _EPW // _GC


def _gather_body(p_hbm, q_hbm, pf_hbm, src_hbm, dst_hbm, h_out, diff_out,
                 sidx, didx, pbuf, qbuf, fd, fs, s0, s1, s2, s3):
    wid = lax.axis_index("s") * _NC + lax.axis_index("c")

    def chunk(j, carry):
        base = wid * _EPW + j * _GC
        pltpu.sync_copy(src_hbm.at[pl.ds(base, _GC)], sidx)
        pltpu.sync_copy(dst_hbm.at[pl.ds(base, _GC)], didx)
        cp1 = pltpu.async_copy(p_hbm.at[didx], pbuf, s0)
        cp2 = pltpu.async_copy(q_hbm.at[sidx], qbuf, s1)
        cp3 = pltpu.async_copy(pf_hbm.at[didx], fd, s2)
        cp4 = pltpu.async_copy(pf_hbm.at[sidx], fs, s3)
        cp1.wait()
        cp2.wait()
        cp3.wait()
        cp4.wait()

        def row(r, c):
            for k in range(HID // 16):
                sl = pl.ds(16 * k, 16)
                pbuf[r, sl] = pbuf[r, sl] + qbuf[r, sl]
            fd[r, :] = fd[r, :] - fs[r, :]
            return c

        lax.fori_loop(0, _GC, row, 0)
        pltpu.sync_copy(pbuf, h_out.at[pl.ds(base, _GC)])
        pltpu.sync_copy(fd, diff_out.at[pl.ds(base, _GC)])
        return carry

    lax.fori_loop(0, _GNCH, chunk, 0)


def _gather_stage(p2, q2, pf16f, src, dst):
    mesh = plsc.VectorSubcoreMesh(core_axis_name="c", subcore_axis_name="s")
    k = pl.kernel(
        _gather_body, mesh=mesh,
        compiler_params=pltpu.CompilerParams(use_tc_tiling_on_sc=False),
        out_type=[
            jax.ShapeDtypeStruct((E, HID), F32),
            jax.ShapeDtypeStruct((E, 16), F32),
        ],
        scratch_types=[
            pltpu.VMEM((_GC,), jnp.int32),
            pltpu.VMEM((_GC,), jnp.int32),
            pltpu.VMEM((_GC, HID), F32),
            pltpu.VMEM((_GC, HID), F32),
            pltpu.VMEM((_GC, 16), F32),
            pltpu.VMEM((_GC, 16), F32),
            pltpu.SemaphoreType.DMA,
            pltpu.SemaphoreType.DMA,
            pltpu.SemaphoreType.DMA,
            pltpu.SemaphoreType.DMA,
        ],
    )
    return k(p2, q2, pf16f, src, dst)


def _scatter_stage(mp, dst):
    return jax.ops.segment_sum(mp, dst, num_segments=NTOT)


def kernel(x, pos, vel, edge_index, edge_attr, params):
    w0r = params['weight_scalar_r'][:, :, 0]
    w1r = params['weight_scalar_r'][:, :, 1]
    w1i = params['weight_scalar_i'][:, :, 1]
    wvr8 = params['weight_vector_r'].reshape(8)
    wvi8 = params['weight_vector_i'].reshape(8)

    (mw1, mb1), (mw2, mb2), (mw3, mb3) = params['message_net']
    wd = mw1[0:D]
    wq = mw1[D:2 * D]
    wdist = mw1[2 * D:2 * D + 1]
    wea = mw1[2 * D + 1:]

    (pv1, pc1), (pv2, pc2), (pv3, pc3) = params['update_pos_net']
    (fu1, fb1), (fu2, fb2), (fu3, fb3) = params['update_feat_net']
    u1a = fu1[0:D]
    u1b = fu1[D:]
    (zv1, zb1), (zv2, zb2), (zv3, zb3) = params['update_vel_net']

    row = lambda b: b.reshape(1, -1)

    x2, p, q = _spectral_x(x, w0r, w1r, w1i, wd, wq)
    center = _center(pos)
    pf16, vf16 = _spectral_v(pos, vel, center, wvr8, wvi8)

    xf = x2.reshape(NTOT, D)
    p2 = p.reshape(NTOT, HID)
    q2 = q.reshape(NTOT, HID)
    pf16f = pf16.reshape(NTOT, 16)
    vf16f = vf16.reshape(NTOT, 16)
    src = edge_index[0]
    dst = edge_index[1]
    ea = edge_attr.reshape(E, D_EDGE)

    h, diff16 = _gather_stage(p2, q2, pf16f, src, dst)

    mp = _edge_mlp(h, diff16, ea, row(wdist.reshape(-1)), wea, row(mb1),
                   mw2, row(mb2), mw3, row(mb3),
                   pv1, row(pc1), pv2, row(pc2), pv3, row(pc3))

    ag = _scatter_stage(mp, dst)

    xn, pn16, vn16 = _node_update(
        xf, ag, pf16f, vf16f,
        u1a, u1b, row(fb1), fu2, row(fb2), fu3, row(fb3),
        zv1, row(zb1), zv2, row(zb2), zv3, row(zb3))

    x_new = xn.reshape(T, N, D)
    pos_new = pn16[:, 0:POS].reshape(T, N, POS)
    vel_new = vn16[:, 0:POS].reshape(T, N, POS)
    return (x_new, pos_new, vel_new)


# speedup vs baseline: 2.8159x; 1.0788x over previous
import functools

import jax
import jax.numpy as jnp
from jax import lax
from jax.experimental import pallas as pl
from jax.experimental.pallas import tpu as pltpu
from jax.experimental.pallas import tpu_sc as plsc

T, N, D = 4, 10000, 128
E = 320000
D_EDGE = 16
POS = 3
HID = 64
NTOT = T * N

F32 = jnp.float32


def _silu(v):
    return v / (1.0 + jnp.exp(-v))


def _dot(a, b):
    return jnp.dot(a, b, preferred_element_type=F32)


def _spectral_x_body(x_ref, w0r_ref, w1r_ref, w1i_ref, wd_ref, wq_ref,
                     x2_ref, p_ref, q_ref):
    x0 = x_ref[0]
    x1 = x_ref[1]
    x2 = x_ref[2]
    x3 = x_ref[3]
    f0 = x0 + x1 + x2 + x3
    a = x0 - x2
    b = x3 - x1
    r0 = _dot(f0, w0r_ref[...])
    r1 = _dot(a, w1r_ref[...]) - _dot(b, w1i_ref[...])
    i1 = _dot(a, w1i_ref[...]) + _dot(b, w1r_ref[...])
    y0 = 0.25 * (r0 + 2.0 * r1)
    y1 = 0.25 * (r0 - 2.0 * i1)
    y2 = 0.25 * (r0 - 2.0 * r1)
    y3 = 0.25 * (r0 + 2.0 * i1)
    o0 = x0 + y0
    o1 = x1 + y1
    o2 = x2 + y2
    o3 = x3 + y3
    x2_ref[0] = o0
    x2_ref[1] = o1
    x2_ref[2] = o2
    x2_ref[3] = o3
    wd = wd_ref[...]
    wq = wq_ref[...]
    p_ref[0] = _dot(o0, wd)
    p_ref[1] = _dot(o1, wd)
    p_ref[2] = _dot(o2, wd)
    p_ref[3] = _dot(o3, wd)
    q_ref[0] = _dot(o0, wq)
    q_ref[1] = _dot(o1, wq)
    q_ref[2] = _dot(o2, wq)
    q_ref[3] = _dot(o3, wq)


def _spectral_x(x, w0r, w1r, w1i, wd, wq):
    nb = 1000
    grid = N // nb
    full = lambda shape: pl.BlockSpec(shape, lambda i: (0,) * len(shape))
    return pl.pallas_call(
        _spectral_x_body,
        grid=(grid,),
        in_specs=[
            pl.BlockSpec((T, nb, D), lambda i: (0, i, 0)),
            full((D, D)), full((D, D)), full((D, D)),
            full((D, HID)), full((D, HID)),
        ],
        out_specs=[
            pl.BlockSpec((T, nb, D), lambda i: (0, i, 0)),
            pl.BlockSpec((T, nb, HID), lambda i: (0, i, 0)),
            pl.BlockSpec((T, nb, HID), lambda i: (0, i, 0)),
        ],
        out_shape=[
            jax.ShapeDtypeStruct((T, N, D), F32),
            jax.ShapeDtypeStruct((T, N, HID), F32),
            jax.ShapeDtypeStruct((T, N, HID), F32),
        ],
    )(x, w0r, w1r, w1i, wd, wq)


def _center_body(pos_ref, out_ref):
    i = pl.program_id(0)

    @pl.when(i == 0)
    def _init():
        out_ref[...] = jnp.zeros_like(out_ref)

    part = jnp.sum(pos_ref[...], axis=1)
    out_ref[...] += part * (1.0 / N)


def _center(pos):
    nb = 1000
    return pl.pallas_call(
        _center_body,
        grid=(N // nb,),
        in_specs=[pl.BlockSpec((T, nb, POS), lambda i: (0, i, 0))],
        out_specs=pl.BlockSpec((T, POS), lambda i: (0, 0)),
        out_shape=jax.ShapeDtypeStruct((T, POS), F32),
    )(pos)


def _spectral_v_body(pos_ref, vel_ref, c_ref, wr_ref, wi_ref, pf_ref, vf_ref):
    center = []
    pc = []
    vv = []
    for t in range(T):
        ct = c_ref[t]
        center.append(ct)
        pc.append(pos_ref[t] - ct)
        vv.append(vel_ref[t])
    f0p = pc[0] + pc[1] + pc[2] + pc[3]
    f0v = vv[0] + vv[1] + vv[2] + vv[3]
    ap = pc[0] - pc[2]
    av = vv[0] - vv[2]
    bp = pc[3] - pc[1]
    bv = vv[3] - vv[1]

    def mix(arr_p, arr_v, w_ref, o, m):
        return arr_p * w_ref[4 * 0 + 2 * o + m] + arr_v * w_ref[4 * 1 + 2 * o + m]

    for o in range(2):
        r0 = mix(f0p, f0v, wr_ref, o, 0)
        r1 = mix(ap, av, wr_ref, o, 1) - mix(bp, bv, wi_ref, o, 1)
        i1 = mix(ap, av, wi_ref, o, 1) + mix(bp, bv, wr_ref, o, 1)
        y = (0.25 * (r0 + 2.0 * r1), 0.25 * (r0 - 2.0 * i1),
             0.25 * (r0 - 2.0 * r1), 0.25 * (r0 + 2.0 * i1))
        for t in range(T):
            if o == 0:
                pf_ref[t, :, 0:POS] = pc[t] + y[t] + center[t]
            else:
                vf_ref[t, :, 0:POS] = vv[t] + y[t]
    nb = pf_ref.shape[1]
    pf_ref[:, :, POS:] = jnp.zeros((T, nb, 16 - POS), F32)
    vf_ref[:, :, POS:] = jnp.zeros((T, nb, 16 - POS), F32)


def _spectral_v(pos, vel, center, wr8, wi8):
    nb = 1000
    return pl.pallas_call(
        _spectral_v_body,
        grid=(N // nb,),
        in_specs=[
            pl.BlockSpec((T, nb, POS), lambda i: (0, i, 0)),
            pl.BlockSpec((T, nb, POS), lambda i: (0, i, 0)),
            pl.BlockSpec((T, POS), lambda i: (0, 0)),
            pl.BlockSpec(memory_space=pltpu.SMEM),
            pl.BlockSpec(memory_space=pltpu.SMEM),
        ],
        out_specs=[
            pl.BlockSpec((T, nb, 16), lambda i: (0, i, 0)),
            pl.BlockSpec((T, nb, 16), lambda i: (0, i, 0)),
        ],
        out_shape=[
            jax.ShapeDtypeStruct((T, N, 16), F32),
            jax.ShapeDtypeStruct((T, N, 16), F32),
        ],
    )(pos, vel, center, wr8, wi8)


def _edge_body(h_ref, diff_ref, ea_ref, wdist_ref, wea_ref, b1_ref,
               w2_ref, b2_ref, w3_ref, b3_ref,
               v1_ref, c1_ref, v2_ref, c2_ref, v3_ref, c3_ref,
               out_ref):
    diff = diff_ref[...]
    dist = jnp.sqrt(jnp.sum(diff * diff, axis=1, keepdims=True) + 1e-12)
    u = h_ref[...] + _dot(ea_ref[...], wea_ref[...]) \
        + dist * wdist_ref[...] + b1_ref[...]
    u = _silu(u)
    u = _silu(_dot(u, w2_ref[...]) + b2_ref[...])
    m = _dot(u, w3_ref[...]) + b3_ref[...]
    p = _silu(_dot(m, v1_ref[...]) + c1_ref[...])
    p = _silu(_dot(p, v2_ref[...]) + c2_ref[...])
    s = _dot(p, v3_ref[...]) + c3_ref[...]
    out_ref[:, 0:HID] = m
    out_ref[:, HID:HID + 16] = diff * s


def _edge_mlp(h, diff16, ea, wdist, wea, b1, w2, b2, w3, b3,
              v1, c1, v2, c2, v3, c3):
    eb = 3200
    grid = E // eb
    full = lambda shape: pl.BlockSpec(shape, lambda i: (0,) * len(shape))
    return pl.pallas_call(
        _edge_body,
        grid=(grid,),
        in_specs=[
            pl.BlockSpec((eb, HID), lambda i: (i, 0)),
            pl.BlockSpec((eb, 16), lambda i: (i, 0)),
            pl.BlockSpec((eb, D_EDGE), lambda i: (i, 0)),
            full((1, HID)), full((D_EDGE, HID)), full((1, HID)),
            full((HID, HID)), full((1, HID)),
            full((HID, HID)), full((1, HID)),
            full((HID, HID)), full((1, HID)),
            full((HID, HID)), full((1, HID)),
            full((HID, 1)), full((1, 1)),
        ],
        out_specs=pl.BlockSpec((eb, HID + 16), lambda i: (i, 0)),
        out_shape=jax.ShapeDtypeStruct((E, HID + 16), F32),
    )(h, diff16, ea, wdist, wea, b1, w2, b2, w3, b3, v1, c1, v2, c2, v3, c3)


def _node_body(xf_ref, ag_ref, pf_ref, vf_ref,
               u1a_ref, u1b_ref, fb1_ref, u2_ref, fb2_ref, u3_ref, fb3_ref,
               z1_ref, zb1_ref, z2_ref, zb2_ref, z3_ref, zb3_ref,
               xn_ref, pn_ref, vn_ref):
    xf = xf_ref[...]
    ag = ag_ref[...]
    am = ag[:, 0:HID]
    ap = ag[:, HID:HID + 16]
    h = _silu(_dot(xf, u1a_ref[...]) + _dot(am, u1b_ref[...]) + fb1_ref[...])
    h = _silu(_dot(h, u2_ref[...]) + fb2_ref[...])
    xn_ref[...] = _dot(h, u3_ref[...]) + fb3_ref[...]
    z = _silu(_dot(xf, z1_ref[...]) + zb1_ref[...])
    z = _silu(_dot(z, z2_ref[...]) + zb2_ref[...])
    s = _dot(z, z3_ref[...]) + zb3_ref[...]
    vn = s * vf_ref[...] + ap
    vn_ref[...] = vn
    pn_ref[...] = pf_ref[...] + vn


def _node_update(xf, ag, pf16, vf16, u1a, u1b, fb1, u2, fb2, u3, fb3,
                 z1, zb1, z2, zb2, z3, zb3):
    nb = 2000
    grid = NTOT // nb
    full = lambda shape: pl.BlockSpec(shape, lambda i: (0,) * len(shape))
    return pl.pallas_call(
        _node_body,
        grid=(grid,),
        in_specs=[
            pl.BlockSpec((nb, D), lambda i: (i, 0)),
            pl.BlockSpec((nb, HID + 16), lambda i: (i, 0)),
            pl.BlockSpec((nb, 16), lambda i: (i, 0)),
            pl.BlockSpec((nb, 16), lambda i: (i, 0)),
            full((D, HID)), full((HID, HID)), full((1, HID)),
            full((HID, HID)), full((1, HID)),
            full((HID, D)), full((1, D)),
            full((D, HID)), full((1, HID)),
            full((HID, HID)), full((1, HID)),
            full((HID, 1)), full((1, 1)),
        ],
        out_specs=[
            pl.BlockSpec((nb, D), lambda i: (i, 0)),
            pl.BlockSpec((nb, 16), lambda i: (i, 0)),
            pl.BlockSpec((nb, 16), lambda i: (i, 0)),
        ],
        out_shape=[
            jax.ShapeDtypeStruct((NTOT, D), F32),
            jax.ShapeDtypeStruct((NTOT, 16), F32),
            jax.ShapeDtypeStruct((NTOT, 16), F32),
        ],
    )(xf, ag, pf16, vf16, u1a, u1b, fb1, u2, fb2, u3, fb3,
      z1, zb1, z2, zb2, z3, zb3)


_NC = 2
_NS = 16
_NW = _NC * _NS
_GC = 80
_EPW = E // _NW
_GNCH = _EPW // _GC


def _gather_body(p_hbm, q_hbm, pf_hbm, src_hbm, dst_hbm, h_out, diff_out,
                 sidx, didx, pbuf, qbuf, fd, fs, s0, s1, s2, s3):
    wid = lax.axis_index("s") * _NC + lax.axis_index("c")

    def chunk(j, carry):
        base = wid * _EPW + j * _GC
        pltpu.sync_copy(src_hbm.at[pl.ds(base, _GC)], sidx)
        pltpu.sync_copy(dst_hbm.at[pl.ds(base, _GC)], didx)
        cp1 = pltpu.async_copy(p_hbm.at[didx], pbuf, s0)
        cp2 = pltpu.async_copy(q_hbm.at[sidx], qbuf, s1)
        cp3 = pltpu.async_copy(pf_hbm.at[didx], fd, s2)
        cp4 = pltpu.async_copy(pf_hbm.at[sidx], fs, s3)
        cp1.wait()
        cp2.wait()
        cp3.wait()
        cp4.wait()

        def row(r, c):
            for k in range(HID // 16):
                sl = pl.ds(16 * k, 16)
                pbuf[r, sl] = pbuf[r, sl] + qbuf[r, sl]
            fd[r, :] = fd[r, :] - fs[r, :]
            return c

        lax.fori_loop(0, _GC, row, 0)
        pltpu.sync_copy(pbuf, h_out.at[pl.ds(base, _GC)])
        pltpu.sync_copy(fd, diff_out.at[pl.ds(base, _GC)])
        return carry

    lax.fori_loop(0, _GNCH, chunk, 0)


def _gather_stage(p2, q2, pf16f, src, dst):
    mesh = plsc.VectorSubcoreMesh(core_axis_name="c", subcore_axis_name="s")
    k = pl.kernel(
        _gather_body, mesh=mesh,
        compiler_params=pltpu.CompilerParams(use_tc_tiling_on_sc=False),
        out_type=[
            jax.ShapeDtypeStruct((E, HID), F32),
            jax.ShapeDtypeStruct((E, 16), F32),
        ],
        scratch_types=[
            pltpu.VMEM((_GC,), jnp.int32),
            pltpu.VMEM((_GC,), jnp.int32),
            pltpu.VMEM((_GC, HID), F32),
            pltpu.VMEM((_GC, HID), F32),
            pltpu.VMEM((_GC, 16), F32),
            pltpu.VMEM((_GC, 16), F32),
            pltpu.SemaphoreType.DMA,
            pltpu.SemaphoreType.DMA,
            pltpu.SemaphoreType.DMA,
            pltpu.SemaphoreType.DMA,
        ],
    )
    return k(p2, q2, pf16f, src, dst)


_SC_HALF = NTOT // _NC
_SC_ROWS = 20480
_SC_RPT = _SC_ROWS // _NS
_SCH = 80
_EPT = E // _NS
_SNCH = _EPT // _SCH
_AGW = HID + 16


def _scatter_body(mp_hbm, dst_hbm, ag_out, didx, mpbuf, zbuf, shared, s0, s1):
    cid = lax.axis_index("c")
    sid = lax.axis_index("s")
    lo = cid * _SC_HALF

    def zrow(r, c):
        for k in range(_AGW // 16):
            zbuf[r, pl.ds(16 * k, 16)] = jnp.zeros((16,), F32)
        return c

    lax.fori_loop(0, 160, zrow, 0)
    for k in range(_SC_RPT // 160):
        pltpu.sync_copy(zbuf, shared.at[pl.ds(sid * _SC_RPT + k * 160, 160)])
    plsc.subcore_barrier()

    def chunk(j, carry):
        base = sid * _EPT + j * _SCH
        pltpu.sync_copy(dst_hbm.at[pl.ds(base, _SCH)], didx)
        cp = pltpu.async_copy(mp_hbm.at[pl.ds(base, _SCH)], mpbuf, s0)

        def fix(i, c):
            v = didx[pl.ds(16 * i, 16)] - lo
            inb = (v >= 0) & (v < _SC_HALF)
            didx[pl.ds(16 * i, 16)] = jnp.where(inb, v, _SC_HALF)
            return c

        lax.fori_loop(0, _SCH // 16, fix, 0)
        cp.wait()
        pltpu.sync_copy(mpbuf, shared.at[didx], add=True)
        return carry

    lax.fori_loop(0, _SNCH, chunk, 0)
    plsc.subcore_barrier()

    for k in range(_SC_RPT // 160):
        row = sid * _SC_RPT + k * 160

        @pl.when(row < _SC_HALF)
        def _cp():
            pltpu.sync_copy(shared.at[pl.ds(row, 160)],
                            ag_out.at[pl.ds(lo + row, 160)])


def _scatter_stage(mp, dst):
    mesh = plsc.VectorSubcoreMesh(core_axis_name="c", subcore_axis_name="s")
    k = pl.kernel(
        _scatter_body, mesh=mesh,
        compiler_params=pltpu.CompilerParams(use_tc_tiling_on_sc=False),
        out_type=jax.ShapeDtypeStruct((NTOT, _AGW), F32),
        scratch_types=[
            pltpu.VMEM((_SCH,), jnp.int32),
            pltpu.VMEM((_SCH, _AGW), F32),
            pltpu.VMEM((160, _AGW), F32),
            pltpu.VMEM_SHARED((_SC_ROWS, _AGW), F32),
            pltpu.SemaphoreType.DMA,
            pltpu.SemaphoreType.DMA,
        ],
    )
    return k(mp, dst)


def kernel(x, pos, vel, edge_index, edge_attr, params):
    w0r = params['weight_scalar_r'][:, :, 0]
    w1r = params['weight_scalar_r'][:, :, 1]
    w1i = params['weight_scalar_i'][:, :, 1]
    wvr8 = params['weight_vector_r'].reshape(8)
    wvi8 = params['weight_vector_i'].reshape(8)

    (mw1, mb1), (mw2, mb2), (mw3, mb3) = params['message_net']
    wd = mw1[0:D]
    wq = mw1[D:2 * D]
    wdist = mw1[2 * D:2 * D + 1]
    wea = mw1[2 * D + 1:]

    (pv1, pc1), (pv2, pc2), (pv3, pc3) = params['update_pos_net']
    (fu1, fb1), (fu2, fb2), (fu3, fb3) = params['update_feat_net']
    u1a = fu1[0:D]
    u1b = fu1[D:]
    (zv1, zb1), (zv2, zb2), (zv3, zb3) = params['update_vel_net']

    row = lambda b: b.reshape(1, -1)

    x2, p, q = _spectral_x(x, w0r, w1r, w1i, wd, wq)
    center = _center(pos)
    pf16, vf16 = _spectral_v(pos, vel, center, wvr8, wvi8)

    xf = x2.reshape(NTOT, D)
    p2 = p.reshape(NTOT, HID)
    q2 = q.reshape(NTOT, HID)
    pf16f = pf16.reshape(NTOT, 16)
    vf16f = vf16.reshape(NTOT, 16)
    src = edge_index[0]
    dst = edge_index[1]
    ea = edge_attr.reshape(E, D_EDGE)

    h, diff16 = _gather_stage(p2, q2, pf16f, src, dst)

    mp = _edge_mlp(h, diff16, ea, row(wdist.reshape(-1)), wea, row(mb1),
                   mw2, row(mb2), mw3, row(mb3),
                   pv1, row(pc1), pv2, row(pc2), pv3, row(pc3))

    ag = _scatter_stage(mp, dst)

    xn, pn16, vn16 = _node_update(
        xf, ag, pf16f, vf16f,
        u1a, u1b, row(fb1), fu2, row(fb2), fu3, row(fb3),
        zv1, row(zb1), zv2, row(zb2), zv3, row(zb3))

    x_new = xn.reshape(T, N, D)
    pos_new = pn16[:, 0:POS].reshape(T, N, POS)
    vel_new = vn16[:, 0:POS].reshape(T, N, POS)
    return (x_new, pos_new, vel_new)
